# baseline (speedup 1.0000x reference)
import functools

import jax
import jax.numpy as jnp
from jax import lax
from jax.experimental import pallas as pl
from jax.experimental.pallas import tpu as pltpu
from jax.experimental.pallas import tpu_sc as plsc

N = 25600
E = 409600
NUMROI = 128
C1 = 32
C2 = 16
NGRAPH = N // NUMROI
EPS = 1e-5

NW = 32
EPW = E // NW
CH = 128
NCH = EPW // CH
ECH = E // CH
NPT = N // 16

_mesh = plsc.VectorSubcoreMesh(core_axis_name="c", subcore_axis_name="s")
_sc_params = pltpu.CompilerParams(use_tc_tiling_on_sc=False)


def _mish(x):
    sp = jnp.maximum(x, 0.0) + jnp.log(1.0 + jnp.exp(-jnp.abs(x)))
    return x * jnp.tanh(sp)


@functools.partial(
    pl.kernel,
    mesh=_mesh,
    compiler_params=_sc_params,
    out_type=jax.ShapeDtypeStruct((2 * N,), jnp.float32),
    scratch_types=[
        pltpu.VMEM((8, CH), jnp.int32),
        pltpu.VMEM((8, CH), jnp.int32),
        pltpu.VMEM((8, CH), jnp.float32),
        pltpu.VMEM((8, CH), jnp.float32),
        pltpu.VMEM((NPT,), jnp.float32),
        pltpu.VMEM_SHARED((N,), jnp.float32),
        pltpu.SemaphoreType.DMA((8,)),
        pltpu.SemaphoreType.DMA((8,)),
    ],
)
def _sc_deg(row_h, col_h, attr_h, degp_h, rowv, colv, av, wv, zbuf, dacc,
            lsem, ssem):
    c = lax.axis_index("c")
    s = lax.axis_index("s")
    wid = c * 16 + s

    def load(i, b):
        base = (wid * NCH + i) * CH
        pltpu.async_copy(row_h.at[pl.ds(base, CH)], rowv.at[b], lsem.at[b])
        pltpu.async_copy(col_h.at[pl.ds(base, CH)], colv.at[b], lsem.at[b])
        pltpu.async_copy(attr_h.at[pl.ds(base, CH)], av.at[b], lsem.at[b])

    def wait_l(b):
        pltpu.make_async_copy(row_h.at[pl.ds(0, CH)], rowv.at[b],
                              lsem.at[b]).wait()
        pltpu.make_async_copy(col_h.at[pl.ds(0, CH)], colv.at[b],
                              lsem.at[b]).wait()
        pltpu.make_async_copy(attr_h.at[pl.ds(0, CH)], av.at[b],
                              lsem.at[b]).wait()

    def maskw(b):
        for g in range(CH // 16):
            sl = pl.ds(g * 16, 16)
            wv[b, sl] = jnp.where(rowv[b, sl] == colv[b, sl], 0.0, av[b, sl])

    def scatter(i, b):
        pltpu.async_copy(wv.at[b], dacc.at[rowv.at[b]], ssem.at[b], add=True)

    def wait_s(b):
        pltpu.make_async_copy(wv.at[b], dacc.at[rowv.at[b]], ssem.at[b]).wait()

    def zb(i, carry):
        zbuf[pl.ds(i * 16, 16)] = jnp.zeros((16,), jnp.float32)
        return carry

    lax.fori_loop(0, NPT // 16, zb, 0)
    pltpu.sync_copy(zbuf, dacc.at[pl.ds(s * NPT, NPT)])
    plsc.subcore_barrier()

    for b in range(8):
        load(b, b)
    for i in range(4):
        wait_l(i)
        maskw(i)
        scatter(i, i)

    def body(k, carry):
        i0 = 4 + k * 8
        for u in range(8):
            i = i0 + u
            b = (4 + u) % 8
            wait_l(b)
            maskw(b)
            scatter(i, b)
            b4 = u % 8
            wait_s(b4)
            load(i + 4, b4)
        return carry

    lax.fori_loop(0, 11, body, 0)

    for i in range(92, 96):
        b = i % 8
        wait_l(b)
        maskw(b)
        scatter(i, b)
        b4 = (i + 4) % 8
        wait_s(b4)
        load(i + 4, b4)
    for i in range(96, 100):
        b = i % 8
        wait_l(b)
        maskw(b)
        scatter(i, b)
    for i in range(92, 100):
        wait_s(i % 8)

    plsc.subcore_barrier()
    pltpu.sync_copy(dacc.at[pl.ds(s * NPT, NPT)], zbuf)
    pltpu.sync_copy(zbuf, degp_h.at[pl.ds(c * N + s * NPT, NPT)])


def _make_sc_scatter(D, stage_u):
    ZR = 400

    scratch = [
        pltpu.VMEM((4, CH), jnp.int32),
        pltpu.VMEM((4, CH), jnp.int32),
        pltpu.VMEM((4, CH), jnp.float32),
        pltpu.VMEM((4, CH, D), jnp.float32),
        pltpu.VMEM((ZR, D), jnp.float32),
        pltpu.VMEM_SHARED((N, D), jnp.float32),
    ]
    if stage_u:
        scratch.append(pltpu.VMEM_SHARED((N, D), jnp.float32))
    scratch += [
        pltpu.SemaphoreType.DMA((4,)),
        pltpu.SemaphoreType.DMA((4,)),
        pltpu.SemaphoreType.DMA((4,)),
    ]
    if stage_u:
        scratch.append(pltpu.SemaphoreType.DMA)

    @functools.partial(
        pl.kernel,
        mesh=_mesh,
        compiler_params=_sc_params,
        out_type=jax.ShapeDtypeStruct((2, N, D), jnp.float32),
        scratch_types=scratch,
    )
    def sck(row_h, col_h, attr_h, u_h, out_h,
            rowv, colv, av, msg, zbuf, acc, *rest):
        if stage_u:
            u_s, lsem, gsem, ssem, fsem = rest
        else:
            u_s = u_h
            lsem, gsem, ssem = rest
        c = lax.axis_index("c")
        s = lax.axis_index("s")
        wid = c * 16 + s

        def load(i, b):
            base = (wid * NCH + i) * CH
            pltpu.async_copy(row_h.at[pl.ds(base, CH)], rowv.at[b],
                             lsem.at[b])
            pltpu.async_copy(col_h.at[pl.ds(base, CH)], colv.at[b],
                             lsem.at[b])
            pltpu.async_copy(attr_h.at[pl.ds(base, CH)], av.at[b],
                             lsem.at[b])

        def wait_l(b):
            pltpu.make_async_copy(row_h.at[pl.ds(0, CH)], rowv.at[b],
                                  lsem.at[b]).wait()
            pltpu.make_async_copy(col_h.at[pl.ds(0, CH)], colv.at[b],
                                  lsem.at[b]).wait()
            pltpu.make_async_copy(attr_h.at[pl.ds(0, CH)], av.at[b],
                                  lsem.at[b]).wait()

        def gather(b):
            pltpu.async_copy(u_s.at[rowv.at[b]], msg.at[b], gsem.at[b])

        def wait_g(b):
            pltpu.make_async_copy(u_s.at[rowv.at[b]], msg.at[b],
                                  gsem.at[b]).wait()

        def scatter(b):
            pltpu.async_copy(msg.at[b], acc.at[colv.at[b]], ssem.at[b],
                             add=True)

        def wait_s(b):
            pltpu.make_async_copy(msg.at[b], acc.at[colv.at[b]],
                                  ssem.at[b]).wait()

        def scale(b):
            for g in range(CH // 16):
                sl = pl.ds(g * 16, 16)
                w16 = jnp.where(rowv[b, sl] == colv[b, sl], 0.0, av[b, sl])
                for l in range(16):
                    e = g * 16 + l
                    sc = w16.at[jnp.full((16,), l, jnp.int32)].get(
                        mode="promise_in_bounds")
                    for q in range(D // 16):
                        msg[b, e, pl.ds(q * 16, 16)] = (
                            msg[b, e, pl.ds(q * 16, 16)] * sc)

        def zb(i, carry):
            for q in range(D // 16):
                zbuf[i, pl.ds(q * 16, 16)] = jnp.zeros((16,), jnp.float32)
            return carry

        stage = pl.ds(s * NPT, NPT)
        if stage_u:
            pltpu.async_copy(u_h.at[stage], u_s.at[stage], fsem)
        lax.fori_loop(0, ZR, zb, 0)
        for t in range(NPT // ZR):
            pltpu.sync_copy(zbuf, acc.at[pl.ds(s * NPT + t * ZR, ZR)])
        if stage_u:
            pltpu.make_async_copy(u_h.at[stage], u_s.at[stage], fsem).wait()
        plsc.subcore_barrier()

        load(0, 0)
        load(1, 1)
        wait_l(0)
        gather(0)
        wait_l(1)
        gather(1)
        for i in range(2):
            wait_g(i)
            scale(i)
            scatter(i)
            b2 = (i + 2) % 4
            load(i + 2, b2)
            wait_l(b2)
            gather(b2)

        def body(k, carry):
            i0 = 2 + k * 4
            for u in range(4):
                i = i0 + u
                b = (2 + u) % 4
                wait_g(b)
                scale(b)
                scatter(b)
                b2 = u % 4
                wait_s(b2)
                load(i + 2, b2)
                wait_l(b2)
                gather(b2)
            return carry

        lax.fori_loop(0, 24, body, 0)

        wait_g(2)
        scale(2)
        scatter(2)
        wait_g(3)
        scale(3)
        scatter(3)
        for b in range(4):
            wait_s(b)

        plsc.subcore_barrier()
        for t in range(NPT // ZR):
            pltpu.sync_copy(acc.at[pl.ds(s * NPT + t * ZR, ZR)], zbuf)
            pltpu.sync_copy(zbuf, out_h.at[c, pl.ds(s * NPT + t * ZR, ZR)])

    return sck


_sc_scatter32 = _make_sc_scatter(C1, stage_u=False)
_sc_scatter16 = _make_sc_scatter(C2, stage_u=True)


_RB = 3200


def _tc_mm(x, W, DO):
    def body(x_ref, W_ref, y_ref):
        y_ref[...] = jnp.dot(x_ref[...], W_ref[...],
                             preferred_element_type=jnp.float32)

    DI = x.shape[1]
    return pl.pallas_call(
        body,
        grid=(N // _RB,),
        in_specs=[
            pl.BlockSpec((_RB, DI), lambda i: (i, 0)),
            pl.BlockSpec((DI, DO), lambda i: (0, 0)),
        ],
        out_specs=pl.BlockSpec((_RB, DO), lambda i: (i, 0)),
        out_shape=jax.ShapeDtypeStruct((N, DO), jnp.float32),
    )(x, W)


def _tc_b1(y1, degpT):
    def body(y_ref, dT_ref, u1_ref, dinv_ref):
        deg = dT_ref[:, 0:1] + dT_ref[:, 1:2]
        dinv = jnp.where(deg > 0.0,
                         lax.rsqrt(jnp.where(deg > 0.0, deg, 1.0)), 0.0)
        u1_ref[...] = dinv * y_ref[...]
        dinv_ref[...] = dinv

    return pl.pallas_call(
        body,
        grid=(N // _RB,),
        in_specs=[
            pl.BlockSpec((_RB, C1), lambda i: (i, 0)),
            pl.BlockSpec((_RB, 2), lambda i: (i, 0)),
        ],
        out_specs=[
            pl.BlockSpec((_RB, C1), lambda i: (i, 0)),
            pl.BlockSpec((_RB, 1), lambda i: (i, 0)),
        ],
        out_shape=[
            jax.ShapeDtypeStruct((N, C1), jnp.float32),
            jax.ShapeDtypeStruct((N, 1), jnp.float32),
        ],
    )(y1, degpT)


def _tc_d(z1, S1p, dinv, b1, W21):
    def body(z1_ref, S_ref, dv_ref, b_ref, W21_ref, h_ref, u2_ref):
        S = S_ref[0] + S_ref[1]
        dv = dv_ref[...]
        h = _mish(z1_ref[...] - dv * S + b_ref[...])
        h_ref[...] = h
        u2_ref[...] = dv * jnp.dot(h, W21_ref[...],
                                   preferred_element_type=jnp.float32)

    return pl.pallas_call(
        body,
        grid=(N // _RB,),
        in_specs=[
            pl.BlockSpec((_RB, C1), lambda i: (i, 0)),
            pl.BlockSpec((2, _RB, C1), lambda i: (0, i, 0)),
            pl.BlockSpec((_RB, 1), lambda i: (i, 0)),
            pl.BlockSpec((1, C1), lambda i: (0, 0)),
            pl.BlockSpec((C1, C2), lambda i: (0, 0)),
        ],
        out_specs=[
            pl.BlockSpec((_RB, C1), lambda i: (i, 0)),
            pl.BlockSpec((_RB, C2), lambda i: (i, 0)),
        ],
        out_shape=[
            jax.ShapeDtypeStruct((N, C1), jnp.float32),
            jax.ShapeDtypeStruct((N, C2), jnp.float32),
        ],
    )(z1, S1p, dinv, b1, W21)


def _tc_f(z2, S2p, dinv, b2, Wro, bro):
    def body(z2_ref, S_ref, dv_ref, b_ref, Wro_ref, bro_ref, out_ref):
        S = S_ref[0] + S_ref[1]
        dv = dv_ref[...]
        h2 = _mish(z2_ref[...] - dv * S + b_ref[...])
        out_ref[...] = _mish(jnp.dot(h2, Wro_ref[...],
                                     preferred_element_type=jnp.float32)
                             + bro_ref[...])

    return pl.pallas_call(
        body,
        grid=(N // _RB,),
        in_specs=[
            pl.BlockSpec((_RB, C2), lambda i: (i, 0)),
            pl.BlockSpec((2, _RB, C2), lambda i: (0, i, 0)),
            pl.BlockSpec((_RB, 1), lambda i: (i, 0)),
            pl.BlockSpec((1, C2), lambda i: (0, 0)),
            pl.BlockSpec((C2, 8), lambda i: (0, 0)),
            pl.BlockSpec((1, 8), lambda i: (0, 0)),
        ],
        out_specs=[pl.BlockSpec((_RB, 8), lambda i: (i, 0))],
        out_shape=[jax.ShapeDtypeStruct((N, 8), jnp.float32)],
    )(z2, S2p, dinv, b2, Wro, bro)[0]


def _tc_head(feat, Wfc1, bfc1, gamma, beta, Wfc2, bfc2):
    def body(f_ref, W1_ref, b1_ref, g_ref, be_ref, W2_ref, b2_ref, out_ref):
        z = jnp.dot(f_ref[...], W1_ref[...],
                    preferred_element_type=jnp.float32) + b1_ref[...]
        mean = jnp.mean(z, axis=0, keepdims=True)
        var = jnp.mean((z - mean) ** 2, axis=0, keepdims=True)
        zn = (z - mean) * lax.rsqrt(var + EPS) * g_ref[...] + be_ref[...]
        out_ref[...] = jnp.dot(_mish(zn), W2_ref[...],
                               preferred_element_type=jnp.float32) + b2_ref[...]

    return pl.pallas_call(
        body,
        out_shape=jax.ShapeDtypeStruct((NGRAPH, 2), jnp.float32),
    )(feat, Wfc1, bfc1, gamma, beta, Wfc2, bfc2)


def kernel(x, edge_index, attr, batch, W10, W11, b1, W20, W21, b2,
           Wro, bro, Wfc1, bfc1, gamma, beta, Wfc2, bfc2):
    row = edge_index[0]
    col = edge_index[1]

    degp = _sc_deg(row, col, attr).reshape(2, N)
    y1 = _tc_mm(x, W11, C1)
    degpT = degp.T

    u1, dinv = _tc_b1(y1, degpT)
    S1p = _sc_scatter32(row, col, attr, u1)
    z1 = _tc_mm(x, W10, C1)
    h, u2 = _tc_d(z1, S1p, dinv, b1.reshape(1, C1), W21)
    S2p = _sc_scatter16(row, col, attr, u2)
    z2 = _tc_mm(h, W20, C2)
    out = _tc_f(z2, S2p, dinv, b2.reshape(1, C2), Wro, bro.reshape(1, 8))
    feat = out.reshape(NGRAPH, NUMROI * 8)
    logits = _tc_head(feat, Wfc1, bfc1.reshape(1, NUMROI),
                      gamma.reshape(1, NUMROI), beta.reshape(1, NUMROI),
                      Wfc2, bfc2.reshape(1, 2))
    return logits


# speedup vs baseline: 1.0164x; 1.0164x over previous
import functools

import jax
import jax.numpy as jnp
from jax import lax
from jax.experimental import pallas as pl
from jax.experimental.pallas import tpu as pltpu
from jax.experimental.pallas import tpu_sc as plsc

N = 25600
E = 409600
NUMROI = 128
C1 = 32
C2 = 16
NGRAPH = N // NUMROI
EPS = 1e-5

NW = 32
EPW = E // NW
CH = 128
NCH = EPW // CH
ECH = E // CH
NPT = N // 16

_mesh = plsc.VectorSubcoreMesh(core_axis_name="c", subcore_axis_name="s")
_sc_params = pltpu.CompilerParams(use_tc_tiling_on_sc=False)


def _mish(x):
    sp = jnp.maximum(x, 0.0) + jnp.log(1.0 + jnp.exp(-jnp.abs(x)))
    return x * jnp.tanh(sp)


@functools.partial(
    pl.kernel,
    mesh=_mesh,
    compiler_params=_sc_params,
    out_type=jax.ShapeDtypeStruct((2 * N,), jnp.float32),
    scratch_types=[
        pltpu.VMEM((8, CH), jnp.int32),
        pltpu.VMEM((8, CH), jnp.int32),
        pltpu.VMEM((8, CH), jnp.float32),
        pltpu.VMEM((8, CH), jnp.float32),
        pltpu.VMEM((NPT,), jnp.float32),
        pltpu.VMEM_SHARED((N,), jnp.float32),
        pltpu.SemaphoreType.DMA((8,)),
        pltpu.SemaphoreType.DMA((8,)),
    ],
)
def _sc_deg(row_h, col_h, attr_h, degp_h, rowv, colv, av, wv, zbuf, dacc,
            lsem, ssem):
    c = lax.axis_index("c")
    s = lax.axis_index("s")
    wid = c * 16 + s

    def load(i, b):
        base = (wid * NCH + i) * CH
        pltpu.async_copy(row_h.at[pl.ds(base, CH)], rowv.at[b], lsem.at[b])
        pltpu.async_copy(col_h.at[pl.ds(base, CH)], colv.at[b], lsem.at[b])
        pltpu.async_copy(attr_h.at[pl.ds(base, CH)], av.at[b], lsem.at[b])

    def wait_l(b):
        pltpu.make_async_copy(row_h.at[pl.ds(0, CH)], rowv.at[b],
                              lsem.at[b]).wait()
        pltpu.make_async_copy(col_h.at[pl.ds(0, CH)], colv.at[b],
                              lsem.at[b]).wait()
        pltpu.make_async_copy(attr_h.at[pl.ds(0, CH)], av.at[b],
                              lsem.at[b]).wait()

    def maskw(b):
        for g in range(CH // 16):
            sl = pl.ds(g * 16, 16)
            wv[b, sl] = jnp.where(rowv[b, sl] == colv[b, sl], 0.0, av[b, sl])

    def scatter(i, b):
        pltpu.async_copy(wv.at[b], dacc.at[rowv.at[b]], ssem.at[b], add=True)

    def wait_s(b):
        pltpu.make_async_copy(wv.at[b], dacc.at[rowv.at[b]], ssem.at[b]).wait()

    def zb(i, carry):
        zbuf[pl.ds(i * 16, 16)] = jnp.zeros((16,), jnp.float32)
        return carry

    lax.fori_loop(0, NPT // 16, zb, 0)
    pltpu.sync_copy(zbuf, dacc.at[pl.ds(s * NPT, NPT)])
    plsc.subcore_barrier()

    for b in range(8):
        load(b, b)
    for i in range(4):
        wait_l(i)
        maskw(i)
        scatter(i, i)

    def body(k, carry):
        i0 = 4 + k * 8
        for u in range(8):
            i = i0 + u
            b = (4 + u) % 8
            wait_l(b)
            maskw(b)
            scatter(i, b)
            b4 = u % 8
            wait_s(b4)
            load(i + 4, b4)
        return carry

    lax.fori_loop(0, 11, body, 0)

    for i in range(92, 96):
        b = i % 8
        wait_l(b)
        maskw(b)
        scatter(i, b)
        b4 = (i + 4) % 8
        wait_s(b4)
        load(i + 4, b4)
    for i in range(96, 100):
        b = i % 8
        wait_l(b)
        maskw(b)
        scatter(i, b)
    for i in range(92, 100):
        wait_s(i % 8)

    plsc.subcore_barrier()
    pltpu.sync_copy(dacc.at[pl.ds(s * NPT, NPT)], zbuf)
    pltpu.sync_copy(zbuf, degp_h.at[pl.ds(c * N + s * NPT, NPT)])


def _make_sc_scatter(D, stage_u):
    ZR = 400

    scratch = [
        pltpu.VMEM((4, CH), jnp.int32),
        pltpu.VMEM((4, CH), jnp.int32),
        pltpu.VMEM((4, CH), jnp.float32),
        pltpu.VMEM((4, CH, D), jnp.float32),
        pltpu.VMEM((ZR, D), jnp.float32),
        pltpu.VMEM_SHARED((N, D), jnp.float32),
    ]
    if stage_u:
        scratch.append(pltpu.VMEM_SHARED((N, D), jnp.float32))
    scratch += [
        pltpu.SemaphoreType.DMA((4,)),
        pltpu.SemaphoreType.DMA((4,)),
        pltpu.SemaphoreType.DMA((4,)),
    ]
    if stage_u:
        scratch.append(pltpu.SemaphoreType.DMA)

    @functools.partial(
        pl.kernel,
        mesh=_mesh,
        compiler_params=_sc_params,
        out_type=jax.ShapeDtypeStruct((2, N, D), jnp.float32),
        scratch_types=scratch,
    )
    def sck(row_h, col_h, attr_h, u_h, out_h,
            rowv, colv, av, msg, zbuf, acc, *rest):
        if stage_u:
            u_s, lsem, gsem, ssem, fsem = rest
        else:
            u_s = u_h
            lsem, gsem, ssem = rest
        c = lax.axis_index("c")
        s = lax.axis_index("s")
        wid = c * 16 + s

        def load(i, b):
            base = (wid * NCH + i) * CH
            pltpu.async_copy(row_h.at[pl.ds(base, CH)], rowv.at[b],
                             lsem.at[b])
            pltpu.async_copy(col_h.at[pl.ds(base, CH)], colv.at[b],
                             lsem.at[b])
            pltpu.async_copy(attr_h.at[pl.ds(base, CH)], av.at[b],
                             lsem.at[b])

        def wait_l(b):
            pltpu.make_async_copy(row_h.at[pl.ds(0, CH)], rowv.at[b],
                                  lsem.at[b]).wait()
            pltpu.make_async_copy(col_h.at[pl.ds(0, CH)], colv.at[b],
                                  lsem.at[b]).wait()
            pltpu.make_async_copy(attr_h.at[pl.ds(0, CH)], av.at[b],
                                  lsem.at[b]).wait()

        def gather(b):
            pltpu.async_copy(u_s.at[rowv.at[b]], msg.at[b], gsem.at[b])

        def wait_g(b):
            pltpu.make_async_copy(u_s.at[rowv.at[b]], msg.at[b],
                                  gsem.at[b]).wait()

        def scatter(b):
            pltpu.async_copy(msg.at[b], acc.at[colv.at[b]], ssem.at[b],
                             add=True)

        def wait_s(b):
            pltpu.make_async_copy(msg.at[b], acc.at[colv.at[b]],
                                  ssem.at[b]).wait()

        def scale(b):
            for g in range(CH // 16):
                sl = pl.ds(g * 16, 16)
                w16 = jnp.where(rowv[b, sl] == colv[b, sl], 0.0, av[b, sl])
                for l in range(16):
                    e = g * 16 + l
                    sc = w16.at[jnp.full((16,), l, jnp.int32)].get(
                        mode="promise_in_bounds")
                    for q in range(D // 16):
                        msg[b, e, pl.ds(q * 16, 16)] = (
                            msg[b, e, pl.ds(q * 16, 16)] * sc)

        def zb(i, carry):
            for q in range(D // 16):
                zbuf[i, pl.ds(q * 16, 16)] = jnp.zeros((16,), jnp.float32)
            return carry

        stage = pl.ds(s * NPT, NPT)
        if stage_u:
            pltpu.async_copy(u_h.at[stage], u_s.at[stage], fsem)
        lax.fori_loop(0, ZR, zb, 0)
        for t in range(NPT // ZR):
            pltpu.sync_copy(zbuf, acc.at[pl.ds(s * NPT + t * ZR, ZR)])
        if stage_u:
            pltpu.make_async_copy(u_h.at[stage], u_s.at[stage], fsem).wait()
        plsc.subcore_barrier()

        load(0, 0)
        load(1, 1)
        wait_l(0)
        gather(0)
        wait_l(1)
        gather(1)
        for i in range(2):
            wait_g(i)
            scale(i)
            scatter(i)
            b2 = (i + 2) % 4
            load(i + 2, b2)
            wait_l(b2)
            gather(b2)

        def body(k, carry):
            i0 = 2 + k * 4
            for u in range(4):
                i = i0 + u
                b = (2 + u) % 4
                wait_g(b)
                scale(b)
                scatter(b)
                b2 = u % 4
                wait_s(b2)
                load(i + 2, b2)
                wait_l(b2)
                gather(b2)
            return carry

        lax.fori_loop(0, 24, body, 0)

        wait_g(2)
        scale(2)
        scatter(2)
        wait_g(3)
        scale(3)
        scatter(3)
        for b in range(4):
            wait_s(b)

        plsc.subcore_barrier()
        for t in range(NPT // ZR):
            pltpu.sync_copy(acc.at[pl.ds(s * NPT + t * ZR, ZR)], zbuf)
            pltpu.sync_copy(zbuf, out_h.at[c, pl.ds(s * NPT + t * ZR, ZR)])

    return sck


_sc_scatter32 = _make_sc_scatter(C1, stage_u=False)
_sc_scatter16 = _make_sc_scatter(C2, stage_u=True)


_RB = 3200


def _tc_b(x, degpT, W10, W11):
    def body(x_ref, dT_ref, W10_ref, W11_ref, z1_ref, u1_ref, dinv_ref):
        deg = dT_ref[:, 0:1] + dT_ref[:, 1:2]
        dinv = jnp.where(deg > 0.0,
                         lax.rsqrt(jnp.where(deg > 0.0, deg, 1.0)), 0.0)
        xb = x_ref[...]
        z1_ref[...] = jnp.dot(xb, W10_ref[...],
                              preferred_element_type=jnp.float32)
        u1_ref[...] = dinv * jnp.dot(xb, W11_ref[...],
                                     preferred_element_type=jnp.float32)
        dinv_ref[...] = dinv

    return pl.pallas_call(
        body,
        grid=(N // _RB,),
        in_specs=[
            pl.BlockSpec((_RB, NUMROI), lambda i: (i, 0)),
            pl.BlockSpec((_RB, 2), lambda i: (i, 0)),
            pl.BlockSpec((NUMROI, C1), lambda i: (0, 0)),
            pl.BlockSpec((NUMROI, C1), lambda i: (0, 0)),
        ],
        out_specs=[
            pl.BlockSpec((_RB, C1), lambda i: (i, 0)),
            pl.BlockSpec((_RB, C1), lambda i: (i, 0)),
            pl.BlockSpec((_RB, 1), lambda i: (i, 0)),
        ],
        out_shape=[
            jax.ShapeDtypeStruct((N, C1), jnp.float32),
            jax.ShapeDtypeStruct((N, C1), jnp.float32),
            jax.ShapeDtypeStruct((N, 1), jnp.float32),
        ],
    )(x, degpT, W10, W11)


def _tc_d(z1, S1p, dinv, b1, W20, W21):
    def body(z1_ref, S_ref, dv_ref, b_ref, W20_ref, W21_ref, z2_ref, u2_ref):
        S = S_ref[0] + S_ref[1]
        dv = dv_ref[...]
        h = _mish(z1_ref[...] - dv * S + b_ref[...])
        z2_ref[...] = jnp.dot(h, W20_ref[...],
                              preferred_element_type=jnp.float32)
        u2_ref[...] = dv * jnp.dot(h, W21_ref[...],
                                   preferred_element_type=jnp.float32)

    return pl.pallas_call(
        body,
        grid=(N // _RB,),
        in_specs=[
            pl.BlockSpec((_RB, C1), lambda i: (i, 0)),
            pl.BlockSpec((2, _RB, C1), lambda i: (0, i, 0)),
            pl.BlockSpec((_RB, 1), lambda i: (i, 0)),
            pl.BlockSpec((1, C1), lambda i: (0, 0)),
            pl.BlockSpec((C1, C2), lambda i: (0, 0)),
            pl.BlockSpec((C1, C2), lambda i: (0, 0)),
        ],
        out_specs=[
            pl.BlockSpec((_RB, C2), lambda i: (i, 0)),
            pl.BlockSpec((_RB, C2), lambda i: (i, 0)),
        ],
        out_shape=[
            jax.ShapeDtypeStruct((N, C2), jnp.float32),
            jax.ShapeDtypeStruct((N, C2), jnp.float32),
        ],
    )(z1, S1p, dinv, b1, W20, W21)


def _tc_f(z2, S2p, dinv, b2, Wro, bro):
    def body(z2_ref, S_ref, dv_ref, b_ref, Wro_ref, bro_ref, out_ref):
        S = S_ref[0] + S_ref[1]
        dv = dv_ref[...]
        h2 = _mish(z2_ref[...] - dv * S + b_ref[...])
        out_ref[...] = _mish(jnp.dot(h2, Wro_ref[...],
                                     preferred_element_type=jnp.float32)
                             + bro_ref[...])

    return pl.pallas_call(
        body,
        grid=(N // _RB,),
        in_specs=[
            pl.BlockSpec((_RB, C2), lambda i: (i, 0)),
            pl.BlockSpec((2, _RB, C2), lambda i: (0, i, 0)),
            pl.BlockSpec((_RB, 1), lambda i: (i, 0)),
            pl.BlockSpec((1, C2), lambda i: (0, 0)),
            pl.BlockSpec((C2, 8), lambda i: (0, 0)),
            pl.BlockSpec((1, 8), lambda i: (0, 0)),
        ],
        out_specs=[pl.BlockSpec((_RB, 8), lambda i: (i, 0))],
        out_shape=[jax.ShapeDtypeStruct((N, 8), jnp.float32)],
    )(z2, S2p, dinv, b2, Wro, bro)[0]


def _tc_head(feat, Wfc1, bfc1, gamma, beta, Wfc2, bfc2):
    def body(f_ref, W1_ref, b1_ref, g_ref, be_ref, W2_ref, b2_ref, out_ref):
        z = jnp.dot(f_ref[...], W1_ref[...],
                    preferred_element_type=jnp.float32) + b1_ref[...]
        mean = jnp.mean(z, axis=0, keepdims=True)
        var = jnp.mean((z - mean) ** 2, axis=0, keepdims=True)
        zn = (z - mean) * lax.rsqrt(var + EPS) * g_ref[...] + be_ref[...]
        out_ref[...] = jnp.dot(_mish(zn), W2_ref[...],
                               preferred_element_type=jnp.float32) + b2_ref[...]

    return pl.pallas_call(
        body,
        out_shape=jax.ShapeDtypeStruct((NGRAPH, 2), jnp.float32),
    )(feat, Wfc1, bfc1, gamma, beta, Wfc2, bfc2)


def kernel(x, edge_index, attr, batch, W10, W11, b1, W20, W21, b2,
           Wro, bro, Wfc1, bfc1, gamma, beta, Wfc2, bfc2):
    row = edge_index[0]
    col = edge_index[1]

    degp = _sc_deg(row, col, attr).reshape(2, N)
    degpT = degp.T

    z1, u1, dinv = _tc_b(x, degpT, W10, W11)
    S1p = _sc_scatter32(row, col, attr, u1)
    z2, u2 = _tc_d(z1, S1p, dinv, b1.reshape(1, C1), W20, W21)
    S2p = _sc_scatter16(row, col, attr, u2)
    out = _tc_f(z2, S2p, dinv, b2.reshape(1, C2), Wro, bro.reshape(1, 8))
    feat = out.reshape(NGRAPH, NUMROI * 8)
    logits = _tc_head(feat, Wfc1, bfc1.reshape(1, NUMROI),
                      gamma.reshape(1, NUMROI), beta.reshape(1, NUMROI),
                      Wfc2, bfc2.reshape(1, 2))
    return logits
